# baseline (speedup 1.0000x reference)
import functools

import jax
import jax.numpy as jnp
from jax import lax
from jax.experimental import pallas as pl
from jax.experimental.pallas import tpu as pltpu
from jax.experimental.pallas import tpu_sc as plsc

_NC = 2
_NS = 16
_NW = _NC * _NS
_SUB = 128
_K = 6
_C = 4
_G = 4
_MEGA = 40
_L = 16


def _sc_gather_transpose(idx2d, W, T, S):
    n_chunks, sub = idx2d.shape
    D = W.shape[1]
    nsub = n_chunks // _NW
    nblk = nsub // _MEGA
    sub_per_t = S // _SUB

    mesh = plsc.VectorSubcoreMesh(core_axis_name="c", subcore_axis_name="s")

    @functools.partial(
        pl.kernel,
        out_type=jax.ShapeDtypeStruct((T, D, S), jnp.float32),
        mesh=mesh,
        scratch_types=[
            pltpu.VMEM((2, _MEGA, _SUB), jnp.int32),
            pltpu.VMEM((_K, _SUB, D), jnp.float32),
            pltpu.VMEM((_C, D, _SUB), jnp.float32),
            pltpu.SemaphoreType.DMA,
            pltpu.SemaphoreType.DMA,
            pltpu.SemaphoreType.DMA,
        ],
        compiler_params=pltpu.CompilerParams(
            use_tc_tiling_on_sc=False, needs_layout_passes=False
        ),
    )
    def k(idx_hbm, w_hbm, out_hbm, idx_v, rows_v, cols_v, isem, gsem, ssem):
        wid = lax.axis_index("s") * _NC + lax.axis_index("c")
        base_sub = wid * nsub

        ridxs = [lax.iota(jnp.int32, _L) + kk * _L for kk in range(_SUB // _L)]
        zero16 = jnp.zeros((_L,), jnp.int32)

        def wait_idx():
            pltpu.make_async_copy(
                idx_hbm.at[pl.ds(base_sub, _MEGA)], idx_v.at[0], isem
            ).wait()

        def wait_gather():
            pltpu.make_async_copy(
                w_hbm.at[idx_v.at[0, 0]], rows_v.at[0], gsem
            ).wait()

        def wait_store():
            pltpu.make_async_copy(
                cols_v.at[0], out_hbm.at[0, :, pl.ds(0, _SUB)], ssem
            ).wait()

        def transpose(sr, sc):
            def dbody(d, carry):
                cidx = zero16 + d
                for kk in range(_SUB // _L):
                    v = plsc.load_gather(rows_v.at[sr], [ridxs[kk], cidx])
                    cols_v[sc, d, pl.ds(kk * _L, _L)] = v
                return carry

            lax.fori_loop(0, D, dbody, 0)

        def process(p):
            sr = p % _K
            sc = p % _C
            wait_gather()
            transpose(sr, sc)
            j = base_sub + p
            t = j // sub_per_t
            i0 = (j % sub_per_t) * _SUB
            pltpu.async_copy(
                cols_v.at[sc], out_hbm.at[t, :, pl.ds(i0, _SUB)], ssem
            )

        pltpu.async_copy(idx_hbm.at[pl.ds(base_sub, _MEGA)], idx_v.at[0], isem)

        def body(jj, carry):
            blk = jj // _MEGA
            q = blk % 2
            r = jj % _MEGA

            @pl.when(r == 0)
            def _():
                wait_idx()

            @pl.when(jnp.logical_and(r == _G, blk + 1 < nblk))
            def _():
                pltpu.async_copy(
                    idx_hbm.at[pl.ds(base_sub + (blk + 1) * _MEGA, _MEGA)],
                    idx_v.at[1 - q],
                    isem,
                )

            pltpu.async_copy(w_hbm.at[idx_v.at[q, r]], rows_v.at[jj % _K], gsem)

            @pl.when(jj >= _G + _C)
            def _():
                wait_store()

            @pl.when(jj >= _G)
            def _():
                process(jj - _G)

            return carry

        lax.fori_loop(0, nsub, body, 0)

        for tt in range(_G):
            wait_store()
            process(nsub - _G + tt)
        for _tt in range(_C):
            wait_store()

    return k(idx2d, W)


def kernel(global_frame_idx, W):
    S, T = global_frame_idx.shape
    D = W.shape[1]
    B = S * T
    idx2d = global_frame_idx.T.astype(jnp.int32).reshape(B // _SUB, _SUB)
    out_phys = _sc_gather_transpose(idx2d, W, T, S)
    return jnp.transpose(out_phys, (2, 0, 1))


# speedup vs baseline: 1.5654x; 1.5654x over previous
import functools

import jax
import jax.numpy as jnp
from jax import lax
from jax.experimental import pallas as pl
from jax.experimental.pallas import tpu as pltpu
from jax.experimental.pallas import tpu_sc as plsc

_NC = 2
_NS = 16
_NW = _NC * _NS
_SUB = 128
_K = 6
_C = 4
_G = 4
_MEGA = 40
_L = 16


def _sc_gather_transpose(idx2d, W, T, S):
    n_chunks, sub = idx2d.shape
    D = W.shape[1]
    nsub = n_chunks // _NW
    nblk = nsub // _MEGA
    sub_per_t = S // _SUB

    mesh = plsc.VectorSubcoreMesh(core_axis_name="c", subcore_axis_name="s")

    @functools.partial(
        pl.kernel,
        out_type=jax.ShapeDtypeStruct((T, D, S), jnp.float32),
        mesh=mesh,
        scratch_types=[
            pltpu.VMEM((2, _MEGA, _SUB), jnp.int32),
            pltpu.VMEM((_K, _SUB, D), jnp.float32),
            pltpu.VMEM((_C, D, _SUB), jnp.float32),
            pltpu.SemaphoreType.DMA,
            pltpu.SemaphoreType.DMA,
            pltpu.SemaphoreType.DMA,
        ],
        compiler_params=pltpu.CompilerParams(
            use_tc_tiling_on_sc=False, needs_layout_passes=False
        ),
    )
    def k(idx_hbm, w_hbm, out_hbm, idx_v, rows_v, cols_v, isem, gsem, ssem):
        wid = lax.axis_index("s") * _NC + lax.axis_index("c")
        base_sub = wid * nsub

        ridxs = [lax.iota(jnp.int32, _L) + kk * _L for kk in range(_SUB // _L)]
        zero16 = jnp.zeros((_L,), jnp.int32)

        def wait_idx():
            pltpu.make_async_copy(
                idx_hbm.at[pl.ds(base_sub, _MEGA)], idx_v.at[0], isem
            ).wait()

        def wait_gather():
            pltpu.make_async_copy(
                w_hbm.at[idx_v.at[0, 0]], rows_v.at[0], gsem
            ).wait()

        def wait_store():
            pltpu.make_async_copy(
                cols_v.at[0], out_hbm.at[0, :, pl.ds(0, _SUB)], ssem
            ).wait()

        def transpose(sr, sc):
            @plsc.parallel_loop(0, D, unroll=4)
            def dbody(d):
                cidx = zero16 + d
                for kk in range(_SUB // _L):
                    v = plsc.load_gather(rows_v.at[sr], [ridxs[kk], cidx])
                    cols_v[sc, d, pl.ds(kk * _L, _L)] = v

        def process(p):
            sr = p % _K
            sc = p % _C
            wait_gather()
            transpose(sr, sc)
            j = base_sub + p
            t = j // sub_per_t
            i0 = (j % sub_per_t) * _SUB
            pltpu.async_copy(
                cols_v.at[sc], out_hbm.at[t, :, pl.ds(i0, _SUB)], ssem
            )

        pltpu.async_copy(idx_hbm.at[pl.ds(base_sub, _MEGA)], idx_v.at[0], isem)

        def body(jj, carry):
            blk = jj // _MEGA
            q = blk % 2
            r = jj % _MEGA

            @pl.when(r == 0)
            def _():
                wait_idx()

            @pl.when(jnp.logical_and(r == _G, blk + 1 < nblk))
            def _():
                pltpu.async_copy(
                    idx_hbm.at[pl.ds(base_sub + (blk + 1) * _MEGA, _MEGA)],
                    idx_v.at[1 - q],
                    isem,
                )

            pltpu.async_copy(w_hbm.at[idx_v.at[q, r]], rows_v.at[jj % _K], gsem)

            @pl.when(jj >= _G + _C)
            def _():
                wait_store()

            @pl.when(jj >= _G)
            def _():
                process(jj - _G)

            return carry

        lax.fori_loop(0, nsub, body, 0)

        for tt in range(_G):
            wait_store()
            process(nsub - _G + tt)
        for _tt in range(_C):
            wait_store()

    return k(idx2d, W)


def kernel(global_frame_idx, W):
    S, T = global_frame_idx.shape
    D = W.shape[1]
    B = S * T
    idx2d = global_frame_idx.T.astype(jnp.int32).reshape(B // _SUB, _SUB)
    out_phys = _sc_gather_transpose(idx2d, W, T, S)
    return jnp.transpose(out_phys, (2, 0, 1))


# speedup vs baseline: 1.9875x; 1.2696x over previous
import functools

import jax
import jax.numpy as jnp
from jax import lax
from jax.experimental import pallas as pl
from jax.experimental.pallas import tpu as pltpu
from jax.experimental.pallas import tpu_sc as plsc

_NC = 2
_NS = 16
_NW = _NC * _NS
_SUB = 128
_K = 12
_G = 6
_MEGA = 40


def _sc_gather(idx2d, W):
    n_chunks, sub = idx2d.shape
    D = W.shape[1]
    B = n_chunks * sub
    nsub = n_chunks // _NW
    nblk = nsub // _MEGA

    mesh = plsc.VectorSubcoreMesh(core_axis_name="c", subcore_axis_name="s")

    @functools.partial(
        pl.kernel,
        out_type=jax.ShapeDtypeStruct((B, D), jnp.float32),
        mesh=mesh,
        scratch_types=[
            pltpu.VMEM((2, _MEGA, _SUB), jnp.int32),
            pltpu.VMEM((_K, _SUB, D), jnp.float32),
            pltpu.SemaphoreType.DMA,
            pltpu.SemaphoreType.DMA,
            pltpu.SemaphoreType.DMA,
        ],
        compiler_params=pltpu.CompilerParams(use_tc_tiling_on_sc=False),
    )
    def k(idx_hbm, w_hbm, out_hbm, idx_v, rows_v, isem, gsem, ssem):
        wid = lax.axis_index("s") * _NC + lax.axis_index("c")
        base_sub = wid * nsub

        def wait_idx():
            pltpu.make_async_copy(
                idx_hbm.at[pl.ds(base_sub, _MEGA)], idx_v.at[0], isem
            ).wait()

        def wait_gather():
            pltpu.make_async_copy(
                w_hbm.at[idx_v.at[0, 0]], rows_v.at[0], gsem
            ).wait()

        def wait_store():
            pltpu.make_async_copy(
                rows_v.at[0], out_hbm.at[pl.ds(0, _SUB)], ssem
            ).wait()

        def fire_store(j, slot):
            pltpu.async_copy(
                rows_v.at[slot],
                out_hbm.at[pl.ds((base_sub + j) * _SUB, _SUB)],
                ssem,
            )

        pltpu.async_copy(idx_hbm.at[pl.ds(base_sub, _MEGA)], idx_v.at[0], isem)

        def body(i, carry):
            s = i % _K
            blk = i // _MEGA
            q = blk % 2
            r = i % _MEGA

            @pl.when(r == 0)
            def _():
                wait_idx()

            @pl.when(jnp.logical_and(r == _G, blk + 1 < nblk))
            def _():
                pltpu.async_copy(
                    idx_hbm.at[pl.ds(base_sub + (blk + 1) * _MEGA, _MEGA)],
                    idx_v.at[1 - q],
                    isem,
                )

            @pl.when(i >= _K)
            def _():
                wait_store()

            pltpu.async_copy(w_hbm.at[idx_v.at[q, r]], rows_v.at[s], gsem)

            @pl.when(i >= _G)
            def _():
                wait_gather()
                fire_store(i - _G, (i - _G) % _K)

            return carry

        lax.fori_loop(0, nsub, body, 0)

        for t in range(_G):
            j = nsub - _G + t
            wait_gather()
            fire_store(j, j % _K)
        for _t in range(_K):
            wait_store()

    return k(idx2d, W)


def _tc_transpose(tmp, T, S, D, bt=8, bi=4096):
    x = tmp.reshape(T, S, D)

    def body(x_ref, o_ref):
        o_ref[...] = jnp.transpose(x_ref[...], (0, 2, 1))

    return pl.pallas_call(
        body,
        grid=(T // bt, S // bi),
        in_specs=[pl.BlockSpec((bt, bi, D), lambda t, b: (t, b, 0))],
        out_specs=pl.BlockSpec((bt, D, bi), lambda t, b: (t, 0, b)),
        out_shape=jax.ShapeDtypeStruct((T, D, S), jnp.float32),
    )(x)


def kernel(global_frame_idx, W):
    S, T = global_frame_idx.shape
    D = W.shape[1]
    B = S * T
    idx2d = global_frame_idx.T.astype(jnp.int32).reshape(B // _SUB, _SUB)
    tmp = _sc_gather(idx2d, W)
    out_phys = _tc_transpose(tmp, T, S, D)
    return jnp.transpose(out_phys, (2, 0, 1))


# speedup vs baseline: 1.9915x; 1.0020x over previous
import functools

import jax
import jax.numpy as jnp
from jax import lax
from jax.experimental import pallas as pl
from jax.experimental.pallas import tpu as pltpu
from jax.experimental.pallas import tpu_sc as plsc

_NC = 2
_NS = 16
_NW = _NC * _NS
_SUB = 128
_K = 12
_G = 6
_MEGA = 40


def _sc_gather(idx2d, W):
    n_chunks, sub = idx2d.shape
    D = W.shape[1]
    B = n_chunks * sub
    nsub = n_chunks // _NW
    nblk = nsub // _MEGA

    mesh = plsc.VectorSubcoreMesh(core_axis_name="c", subcore_axis_name="s")

    @functools.partial(
        pl.kernel,
        out_type=jax.ShapeDtypeStruct((B, D), jnp.float32),
        mesh=mesh,
        scratch_types=[
            pltpu.VMEM((2, _MEGA, _SUB), jnp.int32),
            pltpu.VMEM((_K, _SUB, D), jnp.float32),
            pltpu.SemaphoreType.DMA,
            pltpu.SemaphoreType.DMA,
            pltpu.SemaphoreType.DMA,
        ],
        compiler_params=pltpu.CompilerParams(use_tc_tiling_on_sc=False),
    )
    def k(idx_hbm, w_hbm, out_hbm, idx_v, rows_v, isem, gsem, ssem):
        wid = lax.axis_index("s") * _NC + lax.axis_index("c")
        base_sub = wid * nsub

        def wait_idx():
            pltpu.make_async_copy(
                idx_hbm.at[pl.ds(base_sub, _MEGA)], idx_v.at[0], isem
            ).wait()

        def wait_gather():
            pltpu.make_async_copy(
                w_hbm.at[idx_v.at[0, 0]], rows_v.at[0], gsem
            ).wait()

        def wait_store():
            pltpu.make_async_copy(
                rows_v.at[0], out_hbm.at[pl.ds(0, _SUB)], ssem
            ).wait()

        def fire_store(j, slot):
            pltpu.async_copy(
                rows_v.at[slot],
                out_hbm.at[pl.ds((base_sub + j) * _SUB, _SUB)],
                ssem,
            )

        pltpu.async_copy(idx_hbm.at[pl.ds(base_sub, _MEGA)], idx_v.at[0], isem)

        def body(i, carry):
            s = i % _K
            blk = i // _MEGA
            q = blk % 2
            r = i % _MEGA

            @pl.when(r == 0)
            def _():
                wait_idx()

            @pl.when(jnp.logical_and(r == _G, blk + 1 < nblk))
            def _():
                pltpu.async_copy(
                    idx_hbm.at[pl.ds(base_sub + (blk + 1) * _MEGA, _MEGA)],
                    idx_v.at[1 - q],
                    isem,
                )

            @pl.when(i >= _K)
            def _():
                wait_store()

            pltpu.async_copy(w_hbm.at[idx_v.at[q, r]], rows_v.at[s], gsem)

            @pl.when(i >= _G)
            def _():
                wait_gather()
                fire_store(i - _G, (i - _G) % _K)

            return carry

        lax.fori_loop(0, nsub, body, 0)

        for t in range(_G):
            j = nsub - _G + t
            wait_gather()
            fire_store(j, j % _K)
        for _t in range(_K):
            wait_store()

    return k(idx2d, W)


def _tc_transpose(tmp, T, S, D, bt=1, bi=16384):
    x = tmp.reshape(T, S, D)

    def body(x_ref, o_ref):
        o_ref[...] = jnp.transpose(x_ref[...], (0, 2, 1))

    return pl.pallas_call(
        body,
        grid=(T // bt, S // bi),
        in_specs=[pl.BlockSpec((bt, bi, D), lambda t, b: (t, b, 0))],
        out_specs=pl.BlockSpec((bt, D, bi), lambda t, b: (t, 0, b)),
        out_shape=jax.ShapeDtypeStruct((T, D, S), jnp.float32),
    )(x)


def kernel(global_frame_idx, W):
    S, T = global_frame_idx.shape
    D = W.shape[1]
    B = S * T
    idx2d = global_frame_idx.T.astype(jnp.int32).reshape(B // _SUB, _SUB)
    tmp = _sc_gather(idx2d, W)
    out_phys = _tc_transpose(tmp, T, S, D)
    return jnp.transpose(out_phys, (2, 0, 1))
